# Initial kernel scaffold; baseline (speedup 1.0000x reference)
import jax
import jax.numpy as jnp
from jax.experimental import pallas as pl


def kernel(episode, memory, memory_age, memory_usage, Wq, bq, Wk, bk, Wv, bv):
    raise NotImplementedError("write your pallas kernel here")


# speedup vs baseline: 1.4518x; 1.4518x over previous
import functools
import math

import jax
import jax.numpy as jnp
from jax import lax
from jax.experimental import pallas as pl
from jax.experimental.pallas import tpu as pltpu

B, M, D = 1024, 65536, 256
TM = 2048
NBLK = M // TM
SCALE = 1.0 / math.sqrt(D)


def _pass1_body(episode, Wq, bq, Wk, bk, mem_blk, scores_ref, m_ref, l_ref, q_s):
    i = pl.program_id(0)

    @pl.when(i == 0)
    def _init():
        q = lax.dot_general(episode[...], Wq[...], (((1,), (1,)), ((), ())),
                            preferred_element_type=jnp.float32)
        q_s[...] = (q + bq[...][None, :]) * SCALE
        m_ref[...] = jnp.full((B, 1), -jnp.inf, jnp.float32)
        l_ref[...] = jnp.zeros((B, 1), jnp.float32)

    k = lax.dot_general(mem_blk[...], Wk[...], (((1,), (1,)), ((), ())),
                        preferred_element_type=jnp.float32) + bk[...][None, :]
    s = lax.dot_general(q_s[...], k, (((1,), (1,)), ((), ())),
                        preferred_element_type=jnp.float32)
    scores_ref[...] = s
    bm = jnp.max(s, axis=1, keepdims=True)
    m_old = m_ref[...]
    m_new = jnp.maximum(m_old, bm)
    l_ref[...] = l_ref[...] * jnp.exp(m_old - m_new) + jnp.sum(
        jnp.exp(s - m_new), axis=1, keepdims=True)
    m_ref[...] = m_new


def _pass2_body(scores, mem_blk, m, l, Wv, bv, usage_blk, add1_blk,
                attn_ref, retr_ref, usage_out_ref, racc):
    i = pl.program_id(0)

    @pl.when(i == 0)
    def _init():
        racc[...] = jnp.zeros((B, D), jnp.float32)

    p = jnp.exp(scores[...] - m[...]) * (1.0 / l[...])
    attn_ref[...] = p
    v = lax.dot_general(mem_blk[...], Wv[...], (((1,), (1,)), ((), ())),
                        preferred_element_type=jnp.float32) + bv[...][None, :]
    racc[...] += lax.dot_general(p, v, (((1,), (0,)), ((), ())),
                                 preferred_element_type=jnp.float32)
    usage_out_ref[...] = usage_blk[...] + add1_blk[...] + jnp.sum(
        p, axis=0, keepdims=True)[None]

    @pl.when(i == NBLK - 1)
    def _fin():
        retr_ref[...] = racc[...]


def _attention(episode, memory, usage2, add1_2, Wq, bq, Wk, bk, Wv, bv):
    scores, m, l = pl.pallas_call(
        _pass1_body,
        grid=(NBLK,),
        in_specs=[
            pl.BlockSpec((B, D), lambda i: (0, 0)),
            pl.BlockSpec((D, D), lambda i: (0, 0)),
            pl.BlockSpec((D,), lambda i: (0,)),
            pl.BlockSpec((D, D), lambda i: (0, 0)),
            pl.BlockSpec((D,), lambda i: (0,)),
            pl.BlockSpec((TM, D), lambda i: (i, 0)),
        ],
        out_specs=[
            pl.BlockSpec((B, TM), lambda i: (0, i)),
            pl.BlockSpec((B, 1), lambda i: (0, 0)),
            pl.BlockSpec((B, 1), lambda i: (0, 0)),
        ],
        out_shape=[
            jax.ShapeDtypeStruct((B, M), jnp.float32),
            jax.ShapeDtypeStruct((B, 1), jnp.float32),
            jax.ShapeDtypeStruct((B, 1), jnp.float32),
        ],
        scratch_shapes=[pltpu.VMEM((B, D), jnp.float32)],
    )(episode, Wq, bq, Wk, bk, memory)

    attn, retrieved, usage_out = pl.pallas_call(
        _pass2_body,
        grid=(NBLK,),
        in_specs=[
            pl.BlockSpec((B, TM), lambda i: (0, i)),
            pl.BlockSpec((TM, D), lambda i: (i, 0)),
            pl.BlockSpec((B, 1), lambda i: (0, 0)),
            pl.BlockSpec((B, 1), lambda i: (0, 0)),
            pl.BlockSpec((D, D), lambda i: (0, 0)),
            pl.BlockSpec((D,), lambda i: (0,)),
            pl.BlockSpec((1, 1, TM), lambda i: (i, 0, 0)),
            pl.BlockSpec((1, 1, TM), lambda i: (i, 0, 0)),
        ],
        out_specs=[
            pl.BlockSpec((B, TM), lambda i: (0, i)),
            pl.BlockSpec((B, D), lambda i: (0, 0)),
            pl.BlockSpec((1, 1, TM), lambda i: (i, 0, 0)),
        ],
        out_shape=[
            jax.ShapeDtypeStruct((B, M), jnp.float32),
            jax.ShapeDtypeStruct((B, D), jnp.float32),
            jax.ShapeDtypeStruct((NBLK, 1, TM), jnp.float32),
        ],
        scratch_shapes=[pltpu.VMEM((B, D), jnp.float32)],
        input_output_aliases={0: 0},
    )(scores, memory, m, l, Wv, bv, usage2, add1_2)
    return attn, retrieved, usage_out


def kernel(episode, memory, memory_age, memory_usage, Wq, bq, Wk, bk, Wv, bv):
    _, lru_idx = lax.top_k(-memory_age, B)
    add1 = jnp.zeros((M,), jnp.float32).at[lru_idx].set(1.0)

    usage2 = memory_usage.reshape(NBLK, 1, TM)
    add1_2 = add1.reshape(NBLK, 1, TM)
    attn, retrieved, usage_out = _attention(
        episode, memory, usage2, add1_2, Wq, bq, Wk, bk, Wv, bv)
    new_usage = usage_out.reshape(M)

    new_memory = memory.at[lru_idx].set(episode)
    new_age = memory_age.at[lru_idx].set(jnp.max(memory_age) + 1.0)

    return (retrieved, attn, new_memory, new_age, new_usage)
